# baseline (speedup 1.0000x reference)
import functools

import jax
import jax.numpy as jnp
from jax import lax
from jax.experimental import pallas as pl
from jax.experimental.pallas import tpu as pltpu
from jax.experimental.pallas import tpu_sc as plsc


@functools.lru_cache(maxsize=None)
def _make_sc_kernel(N, B, G, D):
    info = plsc.get_sparse_core_info()
    NC, NS, L = info.num_cores, info.num_subcores, info.num_lanes
    NW = NC * NS
    TCS = B // 128
    UNITS = N * TCS
    BASE = UNITS // NW
    REM = UNITS - NW * BASE
    MAXU = BASE + (1 if REM else 0)
    CI = 64
    NQ = 128 // CI
    NV = D // L
    NH = D // (2 * L)

    assert B % 128 == 0 and D % (2 * L) == 0 and G % 8 == 0 and NQ % 2 == 0

    mesh = plsc.VectorSubcoreMesh(core_axis_name="c", subcore_axis_name="s")

    @functools.partial(
        pl.kernel,
        mesh=mesh,
        out_type=jax.ShapeDtypeStruct((N, B, D), jnp.float32),
        scratch_types=[
            pltpu.VMEM((G, 128), jnp.int32),
            pltpu.VMEM((G * CI, D // 2), jnp.int32),
            pltpu.VMEM((G * CI, D // 2), jnp.int32),
            pltpu.VMEM((CI, D), jnp.float32),
            pltpu.VMEM((CI, D), jnp.float32),
            pltpu.SemaphoreType.DMA,
            pltpu.SemaphoreType.DMA,
            pltpu.SemaphoreType.DMA,
            pltpu.SemaphoreType.DMA,
        ],
        compiler_params=pltpu.CompilerParams(use_tc_tiling_on_sc=False),
    )
    def emb_kernel(table_hbm, idx_hbm, out_hbm,
                   idx_v, rows_a, rows_b, out_a, out_b,
                   sem_a, sem_b, semo_a, semo_b):
        wid = lax.axis_index("s") * NC + lax.axis_index("c")
        mu = BASE + jnp.where(wid < REM, 1, 0)
        s0 = wid * BASE + jnp.minimum(wid, REM)
        inv = jnp.float32(1.0 / G)
        himask = jnp.int32(-65536)
        T = G // 8

        def unit(s):
            u = s0 + s
            return u // TCS, u % TCS

        def stage(s):
            n, tc = unit(s)
            for t in range(T):
                pltpu.sync_copy(idx_hbm.at[(n * T + t) * TCS + tc],
                                idx_v.at[pl.ds(t * 8, 8), :])

        def fire(cq, rows_v, sem):
            for j in range(G):
                pltpu.async_copy(
                    table_hbm.at[idx_v.at[j, pl.ds(cq * CI, CI)]],
                    rows_v.at[pl.ds(j * CI, CI)], sem)

        def drain(cq, rows_v, sem):
            for j in range(G):
                pltpu.make_async_copy(
                    table_hbm.at[idx_v.at[j, pl.ds(cq * CI, CI)]],
                    rows_v.at[pl.ds(j * CI, CI)], sem).wait()

        def out_dst(s, cq):
            n, tc = unit(s)
            return out_hbm.at[n, pl.ds(tc * 128 + cq * CI, CI)]

        def reduce(cq, rows_v, out_v):
            def unpack2(x):
                return (lax.bitcast_convert_type(x << 16, jnp.float32),
                        lax.bitcast_convert_type(x, jnp.float32))

            @plsc.parallel_loop(0, CI, unroll=4)
            def item(c):
                accs = [None] * NV
                for j in range(G):
                    for h in range(NH):
                        x = rows_v[j * CI + c, pl.ds(h * L, L)]
                        lo, hi = unpack2(x)
                        if j == 0:
                            accs[2 * h], accs[2 * h + 1] = lo, hi
                        else:
                            accs[2 * h] = accs[2 * h] + lo
                            accs[2 * h + 1] = accs[2 * h + 1] + hi
                for v in range(NV):
                    out_v[c, pl.ds(v * L, L)] = accs[v] * inv

        stage(0)
        fire(0, rows_a, sem_a)

        def slab(s, carry):
            @pl.when(s < mu)
            def _():
                for cq in range(NQ):
                    par = cq % 2
                    cur, csem = (rows_a, sem_a) if par == 0 else (rows_b, sem_b)
                    out_v, osem = (out_a, semo_a) if par == 0 else (out_b, semo_b)
                    if cq < NQ - 1:
                        nxt, nsem = (rows_b, sem_b) if par == 0 else (rows_a, sem_a)
                        fire(cq + 1, nxt, nsem)
                    drain(cq, cur, csem)
                    if cq == NQ - 1:
                        @pl.when(s + 1 < mu)
                        def _():
                            stage(s + 1)
                            fire(0, rows_a, sem_a)
                    if cq >= 2:
                        pltpu.make_async_copy(out_v, out_dst(s, cq - 2),
                                              osem).wait()
                    else:
                        @pl.when(s >= 1)
                        def _():
                            pltpu.make_async_copy(out_v,
                                                  out_dst(s - 1, NQ + cq - 2),
                                                  osem).wait()
                    reduce(cq, cur, out_v)
                    pltpu.async_copy(out_v, out_dst(s, cq), osem)
            return carry

        lax.fori_loop(0, MAXU, slab, 0)
        pltpu.make_async_copy(out_a, out_dst(mu - 1, NQ - 2), semo_a).wait()
        pltpu.make_async_copy(out_b, out_dst(mu - 1, NQ - 1), semo_b).wait()

    return emb_kernel


def kernel(indices, table):
    B, N, A, T = indices.shape
    G = A * T
    V, D = table.shape
    TCS = B // 128
    idx = indices.reshape(TCS, 128, N, A, T)
    idx = idx.transpose(2, 4, 0, 3, 1).reshape(N * T * TCS, A, 128)
    tbf = table.astype(jnp.bfloat16)
    t32 = jax.lax.bitcast_convert_type(tbf.reshape(V, D // 2, 2), jnp.int32)
    out = _make_sc_kernel(N, B, G, D)(t32, idx)
    out = out.transpose(1, 0, 2)
    out = out.reshape(B, N, D // 32, 2, 16).transpose(0, 1, 2, 4, 3)
    return out.reshape(B, N, D)


# speedup vs baseline: 1.8734x; 1.8734x over previous
import functools

import jax
import jax.numpy as jnp
from jax import lax
from jax.experimental import pallas as pl
from jax.experimental.pallas import tpu as pltpu
from jax.experimental.pallas import tpu_sc as plsc


@functools.lru_cache(maxsize=None)
def _make_sc_kernel(N, B, G, D):
    info = plsc.get_sparse_core_info()
    NC, NS, L = info.num_cores, info.num_subcores, info.num_lanes
    NW = NC * NS
    TCS = B // 128
    UNITS = N * TCS
    BASE = UNITS // NW
    REM = UNITS - NW * BASE
    MAXU = BASE + (1 if REM else 0)
    CI = 64
    NQ = 128 // CI
    NV = D // L
    NH = D // (2 * L)

    assert B % 128 == 0 and D % (2 * L) == 0 and G % 8 == 0 and NQ % 2 == 0

    mesh = plsc.VectorSubcoreMesh(core_axis_name="c", subcore_axis_name="s")

    @functools.partial(
        pl.kernel,
        mesh=mesh,
        out_type=jax.ShapeDtypeStruct((N, B, D), jnp.float32),
        scratch_types=[
            pltpu.VMEM((G, 128), jnp.int32),
            pltpu.VMEM((G * CI, D // 2), jnp.int32),
            pltpu.VMEM((G * CI, D // 2), jnp.int32),
            pltpu.VMEM((CI, D), jnp.float32),
            pltpu.VMEM((CI, D), jnp.float32),
            pltpu.SemaphoreType.DMA,
            pltpu.SemaphoreType.DMA,
            pltpu.SemaphoreType.DMA,
            pltpu.SemaphoreType.DMA,
        ],
        compiler_params=pltpu.CompilerParams(use_tc_tiling_on_sc=False),
    )
    def emb_kernel(table_hbm, idx_hbm, out_hbm,
                   idx_v, rows_a, rows_b, out_a, out_b,
                   sem_a, sem_b, semo_a, semo_b):
        wid = lax.axis_index("s") * NC + lax.axis_index("c")
        mu = BASE + jnp.where(wid < REM, 1, 0)
        s0 = wid * BASE + jnp.minimum(wid, REM)
        inv = jnp.float32(1.0 / G)
        himask = jnp.int32(-65536)
        T = G // 8

        def unit(s):
            u = s0 + s
            return u // TCS, u % TCS

        def stage(s):
            n, tc = unit(s)
            for t in range(T):
                pltpu.sync_copy(idx_hbm.at[(n * T + t) * TCS + tc],
                                idx_v.at[pl.ds(t * 8, 8), :])

        def fire(cq, rows_v, sem):
            for j in range(G):
                pltpu.async_copy(
                    table_hbm.at[idx_v.at[j, pl.ds(cq * CI, CI)]],
                    rows_v.at[pl.ds(j * CI, CI)], sem)

        def drain(cq, rows_v, sem):
            for j in range(G):
                pltpu.make_async_copy(
                    table_hbm.at[idx_v.at[j, pl.ds(cq * CI, CI)]],
                    rows_v.at[pl.ds(j * CI, CI)], sem).wait()

        def out_dst(s, cq):
            n, tc = unit(s)
            return out_hbm.at[n, pl.ds(tc * 128 + cq * CI, CI)]

        def reduce(cq, rows_v, out_v):
            def unpack2(x):
                return (lax.bitcast_convert_type(x << 16, jnp.float32),
                        lax.bitcast_convert_type(x, jnp.float32))

            @plsc.parallel_loop(0, CI, unroll=4)
            def item(c):
                accs = [None] * NV
                for j in range(G):
                    for h in range(NH):
                        x = rows_v[j * CI + c, pl.ds(h * L, L)]
                        lo, hi = unpack2(x)
                        if j == 0:
                            accs[2 * h], accs[2 * h + 1] = lo, hi
                        else:
                            accs[2 * h] = accs[2 * h] + lo
                            accs[2 * h + 1] = accs[2 * h + 1] + hi
                for v in range(NV):
                    out_v[c, pl.ds(v * L, L)] = accs[v] * inv

        stage(0)
        fire(0, rows_a, sem_a)

        def slab(s, carry):
            @pl.when(s < mu)
            def _():
                for cq in range(NQ):
                    par = cq % 2
                    cur, csem = (rows_a, sem_a) if par == 0 else (rows_b, sem_b)
                    out_v, osem = (out_a, semo_a) if par == 0 else (out_b, semo_b)
                    if cq < NQ - 1:
                        nxt, nsem = (rows_b, sem_b) if par == 0 else (rows_a, sem_a)
                        fire(cq + 1, nxt, nsem)
                    drain(cq, cur, csem)
                    if cq == NQ - 1:
                        @pl.when(s + 1 < mu)
                        def _():
                            stage(s + 1)
                            fire(0, rows_a, sem_a)
                    if cq >= 2:
                        pltpu.make_async_copy(out_v, out_dst(s, cq - 2),
                                              osem).wait()
                    else:
                        @pl.when(s >= 1)
                        def _():
                            pltpu.make_async_copy(out_v,
                                                  out_dst(s - 1, NQ + cq - 2),
                                                  osem).wait()
                    reduce(cq, cur, out_v)
                    pltpu.async_copy(out_v, out_dst(s, cq), osem)
            return carry

        lax.fori_loop(0, MAXU, slab, 0)
        pltpu.make_async_copy(out_a, out_dst(mu - 1, NQ - 2), semo_a).wait()
        pltpu.make_async_copy(out_b, out_dst(mu - 1, NQ - 1), semo_b).wait()

    return emb_kernel


def kernel(indices, table):
    B, N, A, T = indices.shape
    G = A * T
    V, D = table.shape
    TCS = B // 128
    idx = indices.reshape(TCS, 128, N, A, T)
    idx = idx.transpose(2, 4, 0, 3, 1).reshape(N * T * TCS, A, 128)
    tbf = table.astype(jnp.bfloat16)
    tbf = tbf.reshape(V, D // 32, 2, 16).transpose(0, 1, 3, 2)
    t32 = jax.lax.bitcast_convert_type(
        tbf.reshape(V, D // 2, 2), jnp.int32)
    out = _make_sc_kernel(N, B, G, D)(t32, idx)
    return out.transpose(1, 0, 2)
